# Initial kernel scaffold; baseline (speedup 1.0000x reference)
import jax
import jax.numpy as jnp
from jax.experimental import pallas as pl


def kernel(feat, user_ids, item_ids, edge_index, W_src, b_src, W_dst, b_dst):
    raise NotImplementedError("write your pallas kernel here")


# speedup vs baseline: 4.1149x; 4.1149x over previous
import functools
import math

import jax
import jax.numpy as jnp
from jax import lax
from jax.experimental import pallas as pl
from jax.experimental.pallas import tpu as pltpu
from jax.experimental.pallas import tpu_sc as plsc

D = 128
N_USER = 5000
N_ITEM = 5000
N_EDGES = 320000
NPAD = 5120
HALF = 160
CHUNK = 160
E_PER_W = N_EDGES // 32
E_PER_T = N_EDGES // 16
INV_SQRT_D = 1.0 / math.sqrt(128.0)

_mesh = plsc.VectorSubcoreMesh(core_axis_name="c", subcore_axis_name="s")
_sc_params = pltpu.CompilerParams(needs_layout_passes=False)


@functools.partial(
    pl.kernel,
    mesh=_mesh,
    out_type=[
        jax.ShapeDtypeStruct((NPAD, D), jnp.float32),
        jax.ShapeDtypeStruct((NPAD, D), jnp.float32),
        jax.ShapeDtypeStruct((N_EDGES,), jnp.float32),
    ],
    scratch_types=[
        pltpu.VMEM_SHARED((NPAD, D), jnp.float32),
        pltpu.VMEM_SHARED((NPAD, D), jnp.float32),
        pltpu.VMEM((CHUNK, D), jnp.float32),
        pltpu.VMEM((CHUNK, D), jnp.float32),
        pltpu.VMEM((HALF,), jnp.int32),
        pltpu.VMEM((CHUNK,), jnp.int32),
        pltpu.VMEM((CHUNK,), jnp.int32),
        pltpu.VMEM((CHUNK,), jnp.float32),
        pltpu.SemaphoreType.DMA,
    ],
    compiler_params=_sc_params,
)
def _k1(feat, uids, iids, eu, ev, hs_out, hd_out, s_out,
        hs_sh, hd_sh, rows_a, rows_b, sidx, ub, vb, sb, sem):
    cid = lax.axis_index("c")
    sid = lax.axis_index("s")
    wid = sid * 2 + cid

    for half in range(2):
        base = sid * (2 * HALF) + half * HALF
        pltpu.sync_copy(uids.at[pl.ds(base, HALF)], sidx)
        pltpu.async_copy(feat.at[sidx], rows_a.at[pl.ds(0, HALF)], sem).wait()
        pltpu.sync_copy(rows_a.at[pl.ds(0, HALF)],
                        hs_sh.at[pl.ds(base, HALF)])

        pltpu.sync_copy(iids.at[pl.ds(base, HALF)], sidx)
        pltpu.async_copy(feat.at[sidx], rows_b.at[pl.ds(0, HALF)], sem).wait()
        pltpu.sync_copy(rows_b.at[pl.ds(0, HALF)],
                        hd_sh.at[pl.ds(base, HALF)])

        @pl.when(cid == 0)
        def _():
            pltpu.sync_copy(rows_a.at[pl.ds(0, HALF)],
                            hs_out.at[pl.ds(base, HALF)])
            pltpu.sync_copy(rows_b.at[pl.ds(0, HALF)],
                            hd_out.at[pl.ds(base, HALF)])

    plsc.subcore_barrier()

    ebase = wid * E_PER_W
    lane = lax.iota(jnp.int32, 16)

    def chunk_body(ci, _):
        eb = ebase + ci * CHUNK
        pltpu.sync_copy(eu.at[pl.ds(eb, CHUNK)], ub)
        pltpu.sync_copy(ev.at[pl.ds(eb, CHUNK)], vb)
        pltpu.async_copy(hs_sh.at[ub], rows_a, sem).wait()
        pltpu.async_copy(hd_sh.at[vb], rows_b, sem).wait()

        def grp_body(g, _):
            e0 = g * 16
            svec = jnp.zeros((16,), jnp.float32)
            for r in range(16):
                acc = rows_a[e0 + r, pl.ds(0, 16)] * rows_b[e0 + r, pl.ds(0, 16)]
                for j in range(1, 8):
                    acc = acc + (rows_a[e0 + r, pl.ds(16 * j, 16)] *
                                 rows_b[e0 + r, pl.ds(16 * j, 16)])
                svec = jnp.where(lane == r, jnp.sum(acc), svec)
            sb[pl.ds(e0, 16)] = svec
            return 0

        lax.fori_loop(0, CHUNK // 16, grp_body, 0)
        pltpu.sync_copy(sb, s_out.at[pl.ds(eb, CHUNK)])
        return 0

    lax.fori_loop(0, E_PER_W // CHUNK, chunk_body, 0)


def _k2_body(hs, hd, ws, bs, wd, bd, s, fs_out, fd_out, w_out):
    dn = (((1,), (1,)), ((), ()))
    fs = lax.dot_general(hs[...], ws[...], dn,
                         preferred_element_type=jnp.float32,
                         precision=lax.Precision.HIGHEST)
    fs_out[...] = jnp.maximum(fs + bs[...], 0.0)
    fd = lax.dot_general(hd[...], wd[...], dn,
                         preferred_element_type=jnp.float32,
                         precision=lax.Precision.HIGHEST)
    fd_out[...] = jnp.maximum(fd + bd[...], 0.0)

    sv = s[...] * INV_SQRT_D
    m = jnp.max(sv)
    e = jnp.exp(sv - m)
    w_out[...] = e / jnp.sum(e)


_k2 = pl.pallas_call(
    _k2_body,
    out_shape=[
        jax.ShapeDtypeStruct((NPAD, D), jnp.float32),
        jax.ShapeDtypeStruct((NPAD, D), jnp.float32),
        jax.ShapeDtypeStruct((N_EDGES // D, D), jnp.float32),
    ],
)


@functools.partial(
    pl.kernel,
    mesh=_mesh,
    out_type=[
        jax.ShapeDtypeStruct((NPAD, D), jnp.float32),
        jax.ShapeDtypeStruct((NPAD, D), jnp.float32),
    ],
    scratch_types=[
        pltpu.VMEM_SHARED((NPAD, D), jnp.float32),
        pltpu.VMEM_SHARED((NPAD, D), jnp.float32),
        pltpu.VMEM((CHUNK, D), jnp.float32),
        pltpu.VMEM((CHUNK,), jnp.int32),
        pltpu.VMEM((CHUNK,), jnp.int32),
        pltpu.VMEM((CHUNK,), jnp.float32),
        pltpu.SemaphoreType.DMA,
    ],
    compiler_params=_sc_params,
)
def _k3(eu, ev, w, fs, fd, user_out, item_out,
        tab_sh, acc_sh, rows, ub, vb, wb, sem):
    cid = lax.axis_index("c")
    sid = lax.axis_index("s")

    zv = jnp.zeros((16,), jnp.float32)

    def zrow(i, _):
        for j in range(8):
            rows[i, pl.ds(16 * j, 16)] = zv
        return 0

    lax.fori_loop(0, HALF, zrow, 0)
    for half in range(2):
        base = sid * (2 * HALF) + half * HALF
        pltpu.sync_copy(rows.at[pl.ds(0, HALF)],
                        acc_sh.at[pl.ds(base, HALF)])

    for half in range(2):
        base = sid * (2 * HALF) + half * HALF

        @pl.when(cid == 0)
        def _():
            pltpu.sync_copy(fs.at[pl.ds(base, HALF)],
                            rows.at[pl.ds(0, HALF)])

        @pl.when(cid == 1)
        def _():
            pltpu.sync_copy(fd.at[pl.ds(base, HALF)],
                            rows.at[pl.ds(0, HALF)])

        pltpu.sync_copy(rows.at[pl.ds(0, HALF)],
                        tab_sh.at[pl.ds(base, HALF)])

    plsc.subcore_barrier()

    ebase = sid * E_PER_T

    def do_chunks(gidx, scidx):
        def chunk_body(ci, _):
            eb = ebase + ci * CHUNK
            pltpu.sync_copy(eu.at[pl.ds(eb, CHUNK)], ub)
            pltpu.sync_copy(ev.at[pl.ds(eb, CHUNK)], vb)
            pltpu.sync_copy(w.at[pl.ds(eb, CHUNK)], wb)
            pltpu.async_copy(tab_sh.at[gidx], rows, sem).wait()

            def scale_grp(g, _):
                e0 = g * 16
                wv = wb[pl.ds(e0, 16)]
                for l in range(16):
                    we = wv[l]
                    for j in range(8):
                        rows[e0 + l, pl.ds(16 * j, 16)] = (
                            rows[e0 + l, pl.ds(16 * j, 16)] * we)
                return 0

            lax.fori_loop(0, CHUNK // 16, scale_grp, 0)
            pltpu.sync_copy(rows, acc_sh.at[scidx], add=True)
            return 0

        lax.fori_loop(0, E_PER_T // CHUNK, chunk_body, 0)

    @pl.when(cid == 0)
    def _():
        do_chunks(ub, vb)

    @pl.when(cid == 1)
    def _():
        do_chunks(vb, ub)

    plsc.subcore_barrier()

    for half in range(2):
        base = sid * (2 * HALF) + half * HALF
        pltpu.sync_copy(acc_sh.at[pl.ds(base, HALF)],
                        rows.at[pl.ds(0, HALF)])

        @pl.when(cid == 0)
        def _():
            pltpu.sync_copy(rows.at[pl.ds(0, HALF)],
                            item_out.at[pl.ds(base, HALF)])

        @pl.when(cid == 1)
        def _():
            pltpu.sync_copy(rows.at[pl.ds(0, HALF)],
                            user_out.at[pl.ds(base, HALF)])


def kernel(feat, user_ids, item_ids, edge_index, W_src, b_src, W_dst, b_dst):
    uids = jnp.pad(user_ids, (0, NPAD - N_USER))
    iids = jnp.pad(item_ids, (0, NPAD - N_ITEM))
    eu = edge_index[0]
    ev = edge_index[1]

    h_src, h_dst, s = _k1(feat, uids, iids, eu, ev)

    feat_src, feat_dst, w2d = _k2(
        h_src, h_dst, W_src, b_src.reshape(1, D), W_dst, b_dst.reshape(1, D),
        s.reshape(N_EDGES // D, D))
    w = w2d.reshape(N_EDGES)

    e_user, e_item = _k3(eu, ev, w, feat_src, feat_dst)
    return jnp.concatenate([e_user[:N_USER], e_item[:N_ITEM]], axis=0)
